# Initial kernel scaffold; baseline (speedup 1.0000x reference)
import jax
import jax.numpy as jnp
from jax.experimental import pallas as pl


def kernel(holder, holder_wgt, table, ln_gamma, ln_beta):
    raise NotImplementedError("write your pallas kernel here")


# speedup vs baseline: 2.3648x; 2.3648x over previous
import functools

import jax
import jax.numpy as jnp
from jax import lax
from jax.experimental import pallas as pl
from jax.experimental.pallas import tpu as pltpu
from jax.experimental.pallas import tpu_sc as plsc

_BATCH = 16384
_HIST = 50
_DIM = 64
_EPS = 1e-5

_ROWS = _BATCH * _HIST
_NC, _NS, _L = 2, 16, 16
_NW = _NC * _NS
_BLK = 128
_NBLK_W = _ROWS // (_NW * _BLK)
_UNROLL = 16


def _ln_row(rows_v, w, r, g, b):
    vw = [rows_v[r, pl.ds(16 * k, 16)] * w for k in range(4)]
    s = (vw[0] + vw[1]) + (vw[2] + vw[3])
    q = (vw[0] * vw[0] + vw[1] * vw[1]) + (vw[2] * vw[2] + vw[3] * vw[3])
    mean = jnp.sum(s) * jnp.float32(1.0 / 64.0)
    var = jnp.sum(q) * jnp.float32(1.0 / 64.0) - mean * mean
    xv = jnp.full((16,), var + jnp.float32(_EPS), jnp.float32)
    ii = plsc.bitcast(xv, jnp.int32)
    ii = jnp.int32(0x5F3759DF) - (ii >> 1)
    y = plsc.bitcast(ii, jnp.float32)
    half = jnp.float32(0.5) * xv
    for _ in range(3):
        y = y * (jnp.float32(1.5) - half * y * y)
    for k in range(4):
        rows_v[r, pl.ds(16 * k, 16)] = (vw[k] - mean) * y * g[k] + b[k]


def _sc_body(table_hbm, idx_hbm, wgt_hbm, gamma_hbm, beta_hbm, out_hbm,
             idx_v, wgt_v, rows_v, g_v, b_v, sem):
    wid = lax.axis_index("s") * _NC + lax.axis_index("c")
    blk0 = wid * _NBLK_W
    pltpu.sync_copy(idx_hbm.at[pl.ds(blk0, _NBLK_W)], idx_v)
    pltpu.sync_copy(wgt_hbm.at[pl.ds(blk0, _NBLK_W)], wgt_v)
    pltpu.sync_copy(gamma_hbm, g_v)
    pltpu.sync_copy(beta_hbm, b_v)
    g = [g_v[pl.ds(16 * k, 16)] for k in range(4)]
    b = [b_v[pl.ds(16 * k, 16)] for k in range(4)]

    def blk_body(j, carry):
        pltpu.async_copy(table_hbm.at[idx_v.at[j]], rows_v, sem).wait()

        def row_body(i, c2):
            wv = wgt_v[j, pl.ds(i * _UNROLL, _UNROLL)]
            for rr in range(_UNROLL):
                _ln_row(rows_v, wv[rr], i * _UNROLL + rr, g, b)
            return c2

        lax.fori_loop(0, _BLK // _UNROLL, row_body, 0)
        pltpu.sync_copy(rows_v, out_hbm.at[pl.ds((blk0 + j) * _BLK, _BLK)])
        return carry

    lax.fori_loop(0, _NBLK_W, blk_body, 0)


@jax.jit
def kernel(holder, holder_wgt, table, ln_gamma, ln_beta):
    idx2d = holder.reshape(_ROWS // _BLK, _BLK)
    wgt2d = holder_wgt.reshape(_ROWS // _BLK, _BLK)
    mesh = plsc.VectorSubcoreMesh(core_axis_name="c", subcore_axis_name="s",
                                  num_cores=_NC, num_subcores=_NS)
    run = pl.kernel(
        _sc_body,
        out_type=jax.ShapeDtypeStruct((_ROWS, _DIM), jnp.float32),
        mesh=mesh,
        scratch_types=[
            pltpu.VMEM((_NBLK_W, _BLK), jnp.int32),
            pltpu.VMEM((_NBLK_W, _BLK), jnp.float32),
            pltpu.VMEM((_BLK, _DIM), jnp.float32),
            pltpu.VMEM((_DIM,), jnp.float32),
            pltpu.VMEM((_DIM,), jnp.float32),
            pltpu.SemaphoreType.DMA,
        ],
        compiler_params=pltpu.CompilerParams(needs_layout_passes=False,
                                             use_tc_tiling_on_sc=False),
    )
    out = run(table, idx2d, wgt2d, ln_gamma, ln_beta)
    return out.reshape(_BATCH, _HIST, _DIM)


# speedup vs baseline: 4.1754x; 1.7657x over previous
import functools

import jax
import jax.numpy as jnp
from jax import lax
from jax.experimental import pallas as pl
from jax.experimental.pallas import tpu as pltpu
from jax.experimental.pallas import tpu_sc as plsc

_BATCH = 16384
_HIST = 50
_DIM = 64
_EPS = 1e-5

_ROWS = _BATCH * _HIST
_NC, _NS, _L = 2, 16, 16
_NW = _NC * _NS
_BLK = 128
_NBLK_W = _ROWS // (_NW * _BLK)
_UNROLL = 8


def _ln_row(rows_in, rows_out, wtmp, r, g, b):
    w = wtmp[pl.ds(r * 16, 16)][0]
    v = [rows_in[r, pl.ds(16 * k, 16)] for k in range(4)]
    s = (v[0] + v[1]) + (v[2] + v[3])
    q = (v[0] * v[0] + v[1] * v[1]) + (v[2] * v[2] + v[3] * v[3])
    mean = jnp.sum(s) * jnp.float32(1.0 / 64.0)
    ex2 = jnp.sum(q) * jnp.float32(1.0 / 64.0)
    x = (ex2 - mean * mean) * (w * w) + jnp.float32(_EPS)
    ii = lax.bitcast_convert_type(x, jnp.int32)
    ii = jnp.int32(0x5F3759DF) - (ii >> 1)
    y = lax.bitcast_convert_type(ii, jnp.float32)
    h = jnp.float32(0.5) * x
    for _ in range(3):
        y = y * (jnp.float32(1.5) - h * y * y)
    a = w * y
    c = mean * a
    for k in range(4):
        rows_out[r, pl.ds(16 * k, 16)] = (v[k] * a - c) * g[k] + b[k]


def _sc_body(table_hbm, idx_hbm, wgt_hbm, gamma_hbm, beta_hbm, out_hbm,
             idx_v, wgt_v, wtmp, rows_in, rows_out, g_v, b_v, sem):
    wid = lax.axis_index("s") * _NC + lax.axis_index("c")
    blk0 = wid * _NBLK_W
    pltpu.sync_copy(idx_hbm.at[pl.ds(blk0, _NBLK_W)], idx_v)
    pltpu.sync_copy(wgt_hbm.at[pl.ds(blk0, _NBLK_W)], wgt_v)
    pltpu.sync_copy(gamma_hbm, g_v)
    pltpu.sync_copy(beta_hbm, b_v)
    g = [g_v[pl.ds(16 * k, 16)] for k in range(4)]
    b = [b_v[pl.ds(16 * k, 16)] for k in range(4)]

    lanes16 = lax.iota(jnp.int32, 16) * 16

    def blk_body(j, carry):
        pltpu.async_copy(table_hbm.at[idx_v.at[j]], rows_in, sem).wait()

        @plsc.parallel_loop(0, _BLK // 16, step=1)
        def wstage(gi):
            wv = wgt_v[j, pl.ds(gi * 16, 16)]
            plsc.store_scatter(wtmp, [lanes16 + gi * 256], wv)

        @plsc.parallel_loop(0, _BLK, step=1, unroll=_UNROLL)
        def row_body(r):
            _ln_row(rows_in, rows_out, wtmp, r, g, b)

        pltpu.sync_copy(rows_out, out_hbm.at[pl.ds((blk0 + j) * _BLK, _BLK)])
        return carry

    lax.fori_loop(0, _NBLK_W, blk_body, 0)


@jax.jit
def kernel(holder, holder_wgt, table, ln_gamma, ln_beta):
    idx2d = holder.reshape(_ROWS // _BLK, _BLK)
    wgt2d = holder_wgt.reshape(_ROWS // _BLK, _BLK)
    mesh = plsc.VectorSubcoreMesh(core_axis_name="c", subcore_axis_name="s",
                                  num_cores=_NC, num_subcores=_NS)
    run = pl.kernel(
        _sc_body,
        out_type=jax.ShapeDtypeStruct((_ROWS, _DIM), jnp.float32),
        mesh=mesh,
        scratch_types=[
            pltpu.VMEM((_NBLK_W, _BLK), jnp.int32),
            pltpu.VMEM((_NBLK_W, _BLK), jnp.float32),
            pltpu.VMEM((_BLK * 16,), jnp.float32),
            pltpu.VMEM((_BLK, _DIM), jnp.float32),
            pltpu.VMEM((_BLK, _DIM), jnp.float32),
            pltpu.VMEM((_DIM,), jnp.float32),
            pltpu.VMEM((_DIM,), jnp.float32),
            pltpu.SemaphoreType.DMA,
        ],
        compiler_params=pltpu.CompilerParams(needs_layout_passes=False,
                                             use_tc_tiling_on_sc=False),
    )
    out = run(table, idx2d, wgt2d, ln_gamma, ln_beta)
    return out.reshape(_BATCH, _HIST, _DIM)
